# Initial kernel scaffold; baseline (speedup 1.0000x reference)
import jax
import jax.numpy as jnp
from jax.experimental import pallas as pl


def kernel(x, type_embed, We1, be1, Wn, bn, We2, be2, Wd1, bd1, Wd2, bd2, Wd3, bd3):
    raise NotImplementedError("write your pallas kernel here")


# speedup vs baseline: 50.1405x; 50.1405x over previous
import numpy as np
import jax
import jax.numpy as jnp
from jax.experimental import pallas as pl
from jax.experimental.pallas import tpu as pltpu

B = 1024
ATOMS = 64
REST = 64
NB = 8
TD = 8
HE = 16
OF = 4
CUTOFF = 5.0
EPS_PER = (ATOMS * ATOMS - ATOMS) // 2
PAIRS = ATOMS * ATOMS
BR = 8

_IU, _JU = np.triu_indices(ATOMS, k=1)
_PAIR_IDX = jnp.asarray(_IU * ATOMS + _JU, dtype=jnp.int32)


def _silu(v):
    return v * jax.nn.sigmoid(v)


def _mm(a, b):
    return jax.lax.dot_general(a, b, (((1,), (0,)), ((), ())),
                               preferred_element_type=jnp.float32,
                               precision=jax.lax.Precision.HIGHEST)


def _body(px_ref, py_ref, pz_ref, xr_ref, te_ref, teT_ref,
          We1_ref, be1_ref, Wn_ref, bn_ref, We2_ref, be2_ref,
          Wd1h_ref, Wd1d_ref, bd1_ref, Wd2_ref, bd2_ref, Wd3_ref, bd3_ref,
          out_ref):
    px = px_ref[...]
    py = py_ref[...]
    pz = pz_ref[...]
    dx = px[:, None, :] - px[:, :, None]
    dy = py[:, None, :] - py[:, :, None]
    dz = pz[:, None, :] - pz[:, :, None]
    d = jnp.sqrt(dx * dx + dy * dy + dz * dz + 1e-12)
    u = jnp.clip(d * (1.0 / CUTOFF), 1e-4, 1.0)
    env_over_u = (1.0 - u) * (1.0 - u) * (1.0 + 2.0 * u) / u
    t = jnp.float32(np.pi) * u
    s1 = jnp.sin(t)
    c1 = jnp.cos(t)
    rbf = [s1 * env_over_u]
    s, c = s1, c1
    for _ in range(NB - 1):
        s, c = s * c1 + c * s1, c * c1 - s * s1
        rbf.append(s * env_over_u)

    te = te_ref[...]
    teT = teT_ref[...]

    hs = []
    for ch in range(HE):
        hp = rbf[0] * We1_ref[0, ch]
        for n in range(1, NB):
            hp = hp + rbf[n] * We1_ref[n, ch]
        es = te[:, 0:1] * We1_ref[NB, ch]
        ed = teT[0:1, :] * We1_ref[NB + TD, ch]
        for tt in range(1, TD):
            es = es + te[:, tt:tt + 1] * We1_ref[NB + tt, ch]
            ed = ed + teT[tt:tt + 1, :] * We1_ref[NB + TD + tt, ch]
        hp = hp + es[None, :, :] + ed[None, :, :] + be1_ref[ch]
        hs.append(_silu(hp))

    ii = jax.lax.broadcasted_iota(jnp.int32, (1, ATOMS, ATOMS), 1)
    jj = jax.lax.broadcasted_iota(jnp.int32, (1, ATOMS, ATOMS), 2)
    mask = ii < jj
    nodes = [jnp.sum(jnp.where(mask, h, 0.0), axis=1) for h in hs]

    gates = []
    for ch in range(HE):
        g = nodes[0] * Wn_ref[0, ch]
        for c2 in range(1, HE):
            g = g + nodes[c2] * Wn_ref[c2, ch]
        gates.append(_silu(g + bn_ref[ch]))

    hg = [hs[ch] * gates[ch][:, :, None] for ch in range(HE)]
    acc = _mm(xr_ref[...], Wd1h_ref[...])
    for f in range(OF):
        eo = hg[0] * We2_ref[0, f]
        for ch in range(1, HE):
            eo = eo + hg[ch] * We2_ref[ch, f]
        eo = eo + be2_ref[f]
        acc = acc + _mm(eo.reshape(BR, PAIRS), Wd1d_ref[f])

    z = _silu(acc + bd1_ref[...])
    z = _silu(_mm(z, Wd2_ref[...]) + bd2_ref[...])
    out_ref[...] = _mm(z, Wd3_ref[...]) + bd3_ref[...]


def kernel(x, type_embed, We1, be1, Wn, bn, We2, be2, Wd1, bd1, Wd2, bd2, Wd3, bd3):
    xr = x[:, :REST]
    pos = x[:, REST:].reshape(B, ATOMS, 3)
    px, py, pz = pos[:, :, 0], pos[:, :, 1], pos[:, :, 2]
    teT = type_embed.T
    Wd1h = Wd1[:REST]
    wed = Wd1[REST:].reshape(EPS_PER, OF, 128).transpose(1, 0, 2)
    Wd1d = jnp.zeros((OF, PAIRS, 128), Wd1.dtype).at[:, _PAIR_IDX, :].set(wed)

    row_spec = pl.BlockSpec((BR, ATOMS), lambda i: (i, 0))
    const = lambda shape: pl.BlockSpec(shape, lambda i: (0,) * len(shape))
    smem = pl.BlockSpec(memory_space=pltpu.SMEM)

    out = pl.pallas_call(
        _body,
        grid=(B // BR,),
        in_specs=[
            row_spec, row_spec, row_spec, row_spec,
            const((ATOMS, TD)), const((TD, ATOMS)),
            smem, smem, smem, smem, smem, smem,
            const((REST, 128)), const((OF, PAIRS, 128)),
            const((1, 128)), const((128, 128)), const((1, 128)),
            const((128, 64)), const((1, 64)),
        ],
        out_specs=pl.BlockSpec((BR, 64), lambda i: (i, 0)),
        out_shape=jax.ShapeDtypeStruct((B, 64), jnp.float32),
        compiler_params=pltpu.CompilerParams(
            dimension_semantics=("parallel",)),
    )(px, py, pz, xr, type_embed, teT, We1, be1, Wn, bn, We2, be2,
      Wd1h, Wd1d, bd1.reshape(1, 128), Wd2, bd2.reshape(1, 128),
      Wd3, bd3.reshape(1, 64))
    return out


# speedup vs baseline: 60.0358x; 1.1974x over previous
import numpy as np
import jax
import jax.numpy as jnp
from jax.experimental import pallas as pl
from jax.experimental.pallas import tpu as pltpu

B = 1024
ATOMS = 64
REST = 64
NB = 8
TD = 8
HE = 16
OF = 4
CUTOFF = 5.0
EPS_PER = (ATOMS * ATOMS - ATOMS) // 2
PAIRS = ATOMS * ATOMS
BR = 16

_IU, _JU = np.triu_indices(ATOMS, k=1)
_PAIR_IDX = np.asarray(_IU * ATOMS + _JU, dtype=np.int32)


def _silu(v):
    return v * jax.nn.sigmoid(v)


def _mm(a, b):
    return jax.lax.dot_general(a, b, (((1,), (0,)), ((), ())),
                               preferred_element_type=jnp.float32,
                               precision=jax.lax.Precision.HIGHEST)


def _body(px_ref, py_ref, pz_ref, xr_ref, te_ref, teT_ref,
          We1_ref, be1_ref, Wn_ref, bn_ref, We2_ref, be2_ref,
          Wd1h_ref, Wd1d_ref, bd1_ref, Wd2_ref, bd2_ref, Wd3_ref, bd3_ref,
          out_ref):
    px = px_ref[...]
    py = py_ref[...]
    pz = pz_ref[...]
    dx = px[:, None, :] - px[:, :, None]
    dy = py[:, None, :] - py[:, :, None]
    dz = pz[:, None, :] - pz[:, :, None]
    d = jnp.sqrt(dx * dx + dy * dy + dz * dz + 1e-12)
    u = jnp.clip(d * (1.0 / CUTOFF), 1e-4, 1.0)
    env_over_u = (1.0 - u) * (1.0 - u) * (1.0 + 2.0 * u) / u
    t = jnp.float32(np.pi) * u
    s1 = jnp.sin(t)
    c1 = jnp.cos(t)
    rbf = [s1 * env_over_u]
    s, c = s1, c1
    for _ in range(NB - 1):
        s, c = s * c1 + c * s1, c * c1 - s * s1
        rbf.append(s * env_over_u)

    te = te_ref[...]
    teT = teT_ref[...]

    hs = []
    for ch in range(HE):
        hp = rbf[0] * We1_ref[0, ch]
        for n in range(1, NB):
            hp = hp + rbf[n] * We1_ref[n, ch]
        es = te[:, 0:1] * We1_ref[NB, ch]
        ed = teT[0:1, :] * We1_ref[NB + TD, ch]
        for tt in range(1, TD):
            es = es + te[:, tt:tt + 1] * We1_ref[NB + tt, ch]
            ed = ed + teT[tt:tt + 1, :] * We1_ref[NB + TD + tt, ch]
        hp = hp + es[None, :, :] + ed[None, :, :] + be1_ref[ch]
        hs.append(_silu(hp))

    ii = jax.lax.broadcasted_iota(jnp.int32, (1, ATOMS, ATOMS), 1)
    jj = jax.lax.broadcasted_iota(jnp.int32, (1, ATOMS, ATOMS), 2)
    mask = ii < jj
    nodes = [jnp.sum(jnp.where(mask, h, 0.0), axis=1) for h in hs]

    gates = []
    for ch in range(HE):
        g = nodes[0] * Wn_ref[0, ch]
        for c2 in range(1, HE):
            g = g + nodes[c2] * Wn_ref[c2, ch]
        gates.append(_silu(g + bn_ref[ch]))

    hg = [hs[ch] * gates[ch][:, :, None] for ch in range(HE)]
    acc = _mm(xr_ref[...], Wd1h_ref[...])
    for f in range(OF):
        eo = hg[0] * We2_ref[0, f]
        for ch in range(1, HE):
            eo = eo + hg[ch] * We2_ref[ch, f]
        eo = eo + be2_ref[f]
        acc = acc + _mm(eo.reshape(BR, PAIRS), Wd1d_ref[f])

    z = _silu(acc + bd1_ref[...])
    z = _silu(_mm(z, Wd2_ref[...]) + bd2_ref[...])
    out_ref[...] = _mm(z, Wd3_ref[...]) + bd3_ref[...]


def kernel(x, type_embed, We1, be1, Wn, bn, We2, be2, Wd1, bd1, Wd2, bd2, Wd3, bd3):
    xr = x[:, :REST]
    pos = x[:, REST:].reshape(B, ATOMS, 3)
    px, py, pz = pos[:, :, 0], pos[:, :, 1], pos[:, :, 2]
    teT = type_embed.T
    Wd1h = Wd1[:REST]
    wed = Wd1[REST:].reshape(EPS_PER, OF, 128).transpose(1, 0, 2)
    Wd1d = jnp.zeros((OF, PAIRS, 128), Wd1.dtype).at[:, _PAIR_IDX, :].set(wed)

    row_spec = pl.BlockSpec((BR, ATOMS), lambda i: (i, 0))
    const = lambda shape: pl.BlockSpec(shape, lambda i: (0,) * len(shape))
    smem = pl.BlockSpec(memory_space=pltpu.SMEM)

    out = pl.pallas_call(
        _body,
        grid=(B // BR,),
        in_specs=[
            row_spec, row_spec, row_spec, row_spec,
            const((ATOMS, TD)), const((TD, ATOMS)),
            smem, smem, smem, smem, smem, smem,
            const((REST, 128)), const((OF, PAIRS, 128)),
            const((1, 128)), const((128, 128)), const((1, 128)),
            const((128, 64)), const((1, 64)),
        ],
        out_specs=pl.BlockSpec((BR, 64), lambda i: (i, 0)),
        out_shape=jax.ShapeDtypeStruct((B, 64), jnp.float32),
        compiler_params=pltpu.CompilerParams(
            dimension_semantics=("parallel",)),
    )(px, py, pz, xr, type_embed, teT, We1, be1, Wn, bn, We2, be2,
      Wd1h, Wd1d, bd1.reshape(1, 128), Wd2, bd2.reshape(1, 128),
      Wd3, bd3.reshape(1, 64))
    return out


# speedup vs baseline: 67.2541x; 1.1202x over previous
import numpy as np
import jax
import jax.numpy as jnp
from jax.experimental import pallas as pl
from jax.experimental.pallas import tpu as pltpu

B = 1024
ATOMS = 64
REST = 64
NB = 8
TD = 8
HE = 16
OF = 4
CUTOFF = 5.0
EPS_PER = (ATOMS * ATOMS - ATOMS) // 2
PAIRS = ATOMS * ATOMS
BR = 16

_IU, _JU = np.triu_indices(ATOMS, k=1)
_PAIR_IDX = np.asarray(_IU * ATOMS + _JU, dtype=np.int32)


def _silu(v):
    return v * jax.nn.sigmoid(v)


def _r(v):
    return v.astype(jnp.bfloat16).astype(jnp.float32)


def _mm(a, b):
    return jax.lax.dot_general(a.astype(jnp.bfloat16), b,
                               (((1,), (0,)), ((), ())),
                               preferred_element_type=jnp.float32)


def _body(px_ref, py_ref, pz_ref, xr_ref, te_ref, teT_ref,
          We1_ref, be1_ref, Wn_ref, bn_ref, We2_ref, be2_ref,
          Wd1h_ref, Wd1d_ref, bd1_ref, Wd2_ref, bd2_ref, Wd3_ref, bd3_ref,
          out_ref):
    px = px_ref[...]
    py = py_ref[...]
    pz = pz_ref[...]
    dx = px[:, None, :] - px[:, :, None]
    dy = py[:, None, :] - py[:, :, None]
    dz = pz[:, None, :] - pz[:, :, None]
    d = jnp.sqrt(dx * dx + dy * dy + dz * dz + 1e-12)
    u = jnp.clip(d * (1.0 / CUTOFF), 1e-4, 1.0)
    env_over_u = (1.0 - u) * (1.0 - u) * (1.0 + 2.0 * u) / u
    t = jnp.float32(np.pi) * u
    s1 = jnp.sin(t)
    c1 = jnp.cos(t)
    rbf = [_r(s1 * env_over_u)]
    s, c = s1, c1
    for _ in range(NB - 1):
        s, c = s * c1 + c * s1, c * c1 - s * s1
        rbf.append(_r(s * env_over_u))

    te = te_ref[...]
    teT = teT_ref[...]

    hs = []
    for ch in range(HE):
        hp = rbf[0] * We1_ref[0, ch]
        for n in range(1, NB):
            hp = hp + rbf[n] * We1_ref[n, ch]
        es = te[:, 0:1] * We1_ref[NB, ch]
        ed = teT[0:1, :] * We1_ref[NB + TD, ch]
        for tt in range(1, TD):
            es = es + te[:, tt:tt + 1] * We1_ref[NB + tt, ch]
            ed = ed + teT[tt:tt + 1, :] * We1_ref[NB + TD + tt, ch]
        hp = hp + es[None, :, :] + ed[None, :, :] + be1_ref[ch]
        hs.append(_silu(hp))

    ii = jax.lax.broadcasted_iota(jnp.int32, (1, ATOMS, ATOMS), 1)
    jj = jax.lax.broadcasted_iota(jnp.int32, (1, ATOMS, ATOMS), 2)
    mask = ii < jj
    nodes = [_r(jnp.sum(jnp.where(mask, h, 0.0), axis=1)) for h in hs]

    gates = []
    for ch in range(HE):
        g = nodes[0] * Wn_ref[0, ch]
        for c2 in range(1, HE):
            g = g + nodes[c2] * Wn_ref[c2, ch]
        gates.append(_silu(g + bn_ref[ch]))

    hg = [_r(hs[ch] * gates[ch][:, :, None]) for ch in range(HE)]
    acc = _mm(xr_ref[...], Wd1h_ref[...])
    for f in range(OF):
        eo = hg[0] * We2_ref[0, f]
        for ch in range(1, HE):
            eo = eo + hg[ch] * We2_ref[ch, f]
        eo = eo + be2_ref[f]
        acc = acc + _mm(eo.reshape(BR, PAIRS), Wd1d_ref[f])

    z = _silu(acc + bd1_ref[...])
    z = _silu(_mm(z, Wd2_ref[...]) + bd2_ref[...])
    out_ref[...] = _mm(z, Wd3_ref[...]) + bd3_ref[...]


def kernel(x, type_embed, We1, be1, Wn, bn, We2, be2, Wd1, bd1, Wd2, bd2, Wd3, bd3):
    f32, bf16 = jnp.float32, jnp.bfloat16
    xr = x[:, :REST]
    pos = x[:, REST:].reshape(B, ATOMS, 3)
    px, py, pz = pos[:, :, 0], pos[:, :, 1], pos[:, :, 2]
    te_r = type_embed.astype(bf16).astype(f32)
    teT_r = te_r.T
    We1_r = We1.astype(bf16).astype(f32)
    Wn_r = Wn.astype(bf16).astype(f32)
    We2_r = We2.astype(bf16).astype(f32)
    Wd1h = Wd1[:REST].astype(bf16)
    wed = Wd1[REST:].reshape(EPS_PER, OF, 128).transpose(1, 0, 2)
    Wd1d = jnp.zeros((OF, PAIRS, 128), f32).at[:, _PAIR_IDX, :].set(wed)
    Wd1d = Wd1d.astype(bf16)

    row_spec = pl.BlockSpec((BR, ATOMS), lambda i: (i, 0))
    const = lambda shape: pl.BlockSpec(shape, lambda i: (0,) * len(shape))
    smem = pl.BlockSpec(memory_space=pltpu.SMEM)

    out = pl.pallas_call(
        _body,
        grid=(B // BR,),
        in_specs=[
            row_spec, row_spec, row_spec, row_spec,
            const((ATOMS, TD)), const((TD, ATOMS)),
            smem, smem, smem, smem, smem, smem,
            const((REST, 128)), const((OF, PAIRS, 128)),
            const((1, 128)), const((128, 128)), const((1, 128)),
            const((128, 64)), const((1, 64)),
        ],
        out_specs=pl.BlockSpec((BR, 64), lambda i: (i, 0)),
        out_shape=jax.ShapeDtypeStruct((B, 64), jnp.float32),
        compiler_params=pltpu.CompilerParams(
            dimension_semantics=("parallel",)),
    )(px, py, pz, xr, te_r, teT_r, We1_r, be1, Wn_r, bn, We2_r, be2,
      Wd1h, Wd1d, bd1.reshape(1, 128), Wd2.astype(bf16), bd2.reshape(1, 128),
      Wd3.astype(bf16), bd3.reshape(1, 64))
    return out


# speedup vs baseline: 67.2614x; 1.0001x over previous
import numpy as np
import jax
import jax.numpy as jnp
from jax.experimental import pallas as pl
from jax.experimental.pallas import tpu as pltpu

B = 1024
ATOMS = 64
REST = 64
NB = 8
TD = 8
HE = 16
OF = 4
CUTOFF = 5.0
EPS_PER = (ATOMS * ATOMS - ATOMS) // 2
PAIRS = ATOMS * ATOMS
BR = 16

_IU, _JU = np.triu_indices(ATOMS, k=1)
_PAIR_IDX = np.asarray(_IU * ATOMS + _JU, dtype=np.int32)


def _silu(v):
    return v * jax.nn.sigmoid(v)


def _rne(v):
    u = jax.lax.bitcast_convert_type(v, jnp.uint32)
    r = (u + jnp.uint32(0x7FFF) + ((u >> 16) & jnp.uint32(1))) & jnp.uint32(0xFFFF0000)
    return jax.lax.bitcast_convert_type(r, jnp.float32)


def _r(v):
    return v.astype(jnp.bfloat16).astype(jnp.float32)


def _mm(a, b):
    return jax.lax.dot_general(a.astype(jnp.bfloat16), b,
                               (((1,), (0,)), ((), ())),
                               preferred_element_type=jnp.float32)


def _body(px_ref, py_ref, pz_ref, xr_ref, te_ref, teT_ref,
          We1_ref, be1_ref, Wn_ref, bn_ref, We2_ref, be2_ref,
          Wd1h_ref, Wd1d_ref, bd1_ref, Wd2_ref, bd2_ref, Wd3_ref, bd3_ref,
          out_ref):
    px = px_ref[...]
    py = py_ref[...]
    pz = pz_ref[...]
    dx = px[:, None, :] - px[:, :, None]
    dy = py[:, None, :] - py[:, :, None]
    dz = pz[:, None, :] - pz[:, :, None]
    d = jnp.sqrt(dx * dx + dy * dy + dz * dz + 1e-12)
    u = jnp.clip(d * (1.0 / CUTOFF), 1e-4, 1.0)
    env_over_u = (1.0 - u) * (1.0 - u) * (1.0 + 2.0 * u) / u
    t = jnp.float32(np.pi) * u
    s1 = jnp.sin(t)
    c1 = jnp.cos(t)
    rbf = [_r(s1 * env_over_u)]
    s, c = s1, c1
    for _ in range(NB - 1):
        s, c = s * c1 + c * s1, c * c1 - s * s1
        rbf.append(_r(s * env_over_u))

    te = te_ref[...]
    teT = teT_ref[...]

    hs = []
    for ch in range(HE):
        hp = rbf[0] * We1_ref[0, ch]
        for n in range(1, NB):
            hp = hp + rbf[n] * We1_ref[n, ch]
        es = te[:, 0:1] * We1_ref[NB, ch]
        ed = teT[0:1, :] * We1_ref[NB + TD, ch]
        for tt in range(1, TD):
            es = es + te[:, tt:tt + 1] * We1_ref[NB + tt, ch]
            ed = ed + teT[tt:tt + 1, :] * We1_ref[NB + TD + tt, ch]
        hp = hp + es[None, :, :] + ed[None, :, :] + be1_ref[ch]
        hs.append(_silu(hp))

    ii = jax.lax.broadcasted_iota(jnp.int32, (1, ATOMS, ATOMS), 1)
    jj = jax.lax.broadcasted_iota(jnp.int32, (1, ATOMS, ATOMS), 2)
    mask = ii < jj
    nodes = [_r(jnp.sum(jnp.where(mask, h, 0.0), axis=1)) for h in hs]

    gates = []
    for ch in range(HE):
        g = nodes[0] * Wn_ref[0, ch]
        for c2 in range(1, HE):
            g = g + nodes[c2] * Wn_ref[c2, ch]
        gates.append(_silu(g + bn_ref[ch]))

    hg = [_r(hs[ch] * gates[ch][:, :, None]) for ch in range(HE)]
    acc = _mm(xr_ref[...], Wd1h_ref[...])
    for f in range(OF):
        eo = hg[0] * We2_ref[0, f]
        for ch in range(1, HE):
            eo = eo + hg[ch] * We2_ref[ch, f]
        eo = eo + be2_ref[f]
        acc = acc + _mm(eo.reshape(BR, PAIRS), Wd1d_ref[f])

    z = _silu(acc + bd1_ref[...])
    z = _silu(_mm(z, Wd2_ref[...]) + bd2_ref[...])
    out_ref[...] = _mm(z, Wd3_ref[...]) + bd3_ref[...]


def kernel(x, type_embed, We1, be1, Wn, bn, We2, be2, Wd1, bd1, Wd2, bd2, Wd3, bd3):
    f32, bf16 = jnp.float32, jnp.bfloat16
    xr = x[:, :REST]
    pos = x[:, REST:].reshape(B, ATOMS, 3)
    px, py, pz = pos[:, :, 0], pos[:, :, 1], pos[:, :, 2]
    type_embed = _rne(type_embed)
    We1 = _rne(We1)
    Wn = _rne(Wn)
    We2 = _rne(We2)
    teT = type_embed.T
    Wd1h = Wd1[:REST].astype(bf16)
    wed = Wd1[REST:].reshape(EPS_PER, OF, 128).transpose(1, 0, 2)
    Wd1d = jnp.zeros((OF, PAIRS, 128), f32).at[:, _PAIR_IDX, :].set(wed)
    Wd1d = Wd1d.astype(bf16)

    row_spec = pl.BlockSpec((BR, ATOMS), lambda i: (i, 0))
    const = lambda shape: pl.BlockSpec(shape, lambda i: (0,) * len(shape))
    smem = pl.BlockSpec(memory_space=pltpu.SMEM)

    out = pl.pallas_call(
        _body,
        grid=(B // BR,),
        in_specs=[
            row_spec, row_spec, row_spec, row_spec,
            const((ATOMS, TD)), const((TD, ATOMS)),
            smem, smem, smem, smem, smem, smem,
            const((REST, 128)), const((OF, PAIRS, 128)),
            const((1, 128)), const((128, 128)), const((1, 128)),
            const((128, 64)), const((1, 64)),
        ],
        out_specs=pl.BlockSpec((BR, 64), lambda i: (i, 0)),
        out_shape=jax.ShapeDtypeStruct((B, 64), jnp.float32),
        compiler_params=pltpu.CompilerParams(
            dimension_semantics=("parallel",)),
    )(px, py, pz, xr, type_embed, teT, We1, be1, Wn, bn, We2, be2,
      Wd1h, Wd1d, bd1.reshape(1, 128), Wd2.astype(bf16), bd2.reshape(1, 128),
      Wd3.astype(bf16), bd3.reshape(1, 64))
    return out
